# Initial kernel scaffold; baseline (speedup 1.0000x reference)
import jax
import jax.numpy as jnp
from jax.experimental import pallas as pl


def kernel(h, x, src, dst, distances, W_msg1, b_msg1, W_msg2, b_msg2, W_h1, b_h1, W_h2, b_h2, W_c1, b_c1, W_c2, b_c2):
    raise NotImplementedError("write your pallas kernel here")


# speedup vs baseline: 3.8318x; 3.8318x over previous
import functools

import jax
import jax.numpy as jnp
from jax import lax
from jax.experimental import pallas as pl
from jax.experimental.pallas import tpu as pltpu
from jax.experimental.pallas import tpu_sc as plsc

F32 = jnp.float32
EP = 327680
EB = 4096
L = 16


def _silu(z):
  return z * jax.nn.sigmoid(z)


def _tables_body(h_ref, w1a_ref, w1b_ref, p_ref, q_ref):
  h = h_ref[...]
  p_ref[...] = jnp.dot(h, w1a_ref[...], preferred_element_type=F32)
  q_ref[...] = jnp.dot(h, w1b_ref[...], preferred_element_type=F32)


def _edge_body(E, a_ref, b_ref, d_ref, wd_ref, b1_ref, w2_ref, b2_ref,
               wc1_ref, bc1_ref, wc2_ref, bc2_ref, mo_ref, co_ref):
  i = pl.program_id(0)
  d = d_ref[...]
  m1 = a_ref[...] + b_ref[...] + (d * d) * wd_ref[...] + b1_ref[...]
  mij = _silu(jnp.dot(_silu(m1), w2_ref[...], preferred_element_type=F32)
              + b2_ref[...])
  c1 = _silu(jnp.dot(mij, wc1_ref[...], preferred_element_type=F32)
             + bc1_ref[...])
  coef = jnp.tanh(jnp.sum(c1 * wc2_ref[...], axis=-1, keepdims=True)
                  + bc2_ref[...])
  rid = lax.broadcasted_iota(jnp.int32, (EB, 1), 0) + i * EB
  valid = jnp.where(rid < E, 1.0, 0.0).astype(F32)
  mo_ref[...] = mij * valid
  co_ref[...] = (coef * valid).reshape(EB // 128, 128)


def _tailsum_body(t_ref, o_ref):
  i = pl.program_id(0)

  @pl.when(i == 0)
  def _():
    o_ref[...] = jnp.zeros_like(o_ref)

  o_ref[...] += t_ref[...]


def _node_body(h_ref, xp_ref, p0_ref, p1_ref, tl_ref, wha_ref,
               whb_ref, bh1_ref, wh2_ref, bh2_ref, ho_ref, xo_ref):
  tail = tl_ref[...]
  cnt = tail[:, 3:4]
  cmax = jnp.maximum(cnt, 1.0)
  mi = (p0_ref[...] + p1_ref[...]) / cmax
  h = h_ref[...]
  u = _silu(jnp.dot(h, wha_ref[...], preferred_element_type=F32)
            + jnp.dot(mi, whb_ref[...], preferred_element_type=F32)
            + bh1_ref[...])
  ho_ref[...] = h + jnp.dot(u, wh2_ref[...], preferred_element_type=F32) + bh2_ref[...]
  xo_ref[...] = xp_ref[...] + tail / cmax


def _build_gather(D, nch, k, per, mesh, NC):
  ch = k * per

  @functools.partial(
      pl.kernel,
      out_type=(jax.ShapeDtypeStruct((EP, D), F32),
                jax.ShapeDtypeStruct((EP, D), F32)),
      mesh=mesh,
      scratch_types=[
          [pltpu.VMEM((per,), jnp.int32) for _ in range(k)],
          [pltpu.VMEM((per,), jnp.int32) for _ in range(k)],
          pltpu.VMEM((ch, D), F32),
          pltpu.VMEM((ch, D), F32),
          pltpu.SemaphoreType.DMA,
          pltpu.SemaphoreType.DMA,
      ],
  )
  def gather_k(p_hbm, q_hbm, src_hbm, dst_hbm, oa_hbm, ob_hbm,
               idxs, idxd, bufa, bufb, sema, semb):
    wid = lax.axis_index("s") * NC + lax.axis_index("c")

    def chunk(j, carry):
      ebase = wid * (nch * ch) + j * ch
      for jj in range(k):
        pltpu.sync_copy(src_hbm.at[pl.ds(ebase + jj * per, per)], idxs[jj])
        pltpu.sync_copy(dst_hbm.at[pl.ds(ebase + jj * per, per)], idxd[jj])
      cps = []
      for jj in range(k):
        cps.append(pltpu.async_copy(p_hbm.at[idxs[jj]],
                                    bufa.at[pl.ds(jj * per, per)], sema))
      for jj in range(k):
        cps.append(pltpu.async_copy(q_hbm.at[idxd[jj]],
                                    bufb.at[pl.ds(jj * per, per)], semb))
      for cp in cps:
        cp.wait()
      pltpu.sync_copy(bufa, oa_hbm.at[pl.ds(ebase, ch)])
      pltpu.sync_copy(bufb, ob_hbm.at[pl.ds(ebase, ch)])
      return carry

    lax.fori_loop(0, nch, chunk, 0)

  return gather_k


def _build_scatter(N, E, D, nch, k, per, mesh, NC, NS):
  ch = k * per
  ra = (N // NS) // 8 * 8
  rb = N - (NS - 1) * ra

  @functools.partial(
      pl.kernel,
      out_type=jax.ShapeDtypeStruct((NC * N, D), F32),
      mesh=mesh,
      scratch_types=[
          [pltpu.VMEM((per,), jnp.int32) for _ in range(k)],
          pltpu.VMEM((ch, D), F32),
          pltpu.VMEM_SHARED((N, D), F32),
      ],
      compiler_params=pltpu.CompilerParams(needs_layout_passes=False),
  )
  def scatter_k(e_hbm, src_hbm, z_hbm, out_hbm, idxs, buf, acc_sh):
    c = lax.axis_index("c")
    s = lax.axis_index("s")
    wid = s * NC + c

    @pl.when(s < NS - 1)
    def _():
      pltpu.sync_copy(z_hbm.at[pl.ds(s * ra, ra)], acc_sh.at[pl.ds(s * ra, ra)])

    @pl.when(s == NS - 1)
    def _():
      pltpu.sync_copy(z_hbm.at[pl.ds((NS - 1) * ra, rb)],
                      acc_sh.at[pl.ds((NS - 1) * ra, rb)])

    plsc.subcore_barrier()

    def chunk(j, carry):
      ebase = wid * (nch * ch) + j * ch
      for jj in range(k):
        pltpu.sync_copy(src_hbm.at[pl.ds(ebase + jj * per, per)], idxs[jj])
      pltpu.sync_copy(e_hbm.at[pl.ds(ebase, ch)], buf)
      for jj in range(k):
        pltpu.sync_copy(buf.at[pl.ds(jj * per, per)], acc_sh.at[idxs[jj]],
                        add=True)
      return carry

    lax.fori_loop(0, nch, chunk, 0)
    plsc.subcore_barrier()

    @pl.when(s < NS - 1)
    def _():
      pltpu.sync_copy(acc_sh.at[pl.ds(s * ra, ra)],
                      out_hbm.at[pl.ds(c * N + s * ra, ra)])

    @pl.when(s == NS - 1)
    def _():
      pltpu.sync_copy(acc_sh.at[pl.ds((NS - 1) * ra, rb)],
                      out_hbm.at[pl.ds(c * N + (NS - 1) * ra, rb)])

  return scatter_k


def _build_tail_scatter(N, E, nch, ch, mesh, NC, NS):
  TR = ((N + 31) // 32 + 7) // 8 * 8

  @functools.partial(
      pl.kernel,
      out_type=jax.ShapeDtypeStruct((NC * NS * TR, 128), F32),
      mesh=mesh,
      scratch_types=[
          pltpu.VMEM((ch,), jnp.int32),
          pltpu.VMEM((ch,), jnp.int32),
          pltpu.VMEM((ch,), F32),
          pltpu.VMEM((N * 4,), F32),
          pltpu.VMEM((TR, 128), F32),
      ],
      compiler_params=pltpu.CompilerParams(needs_layout_passes=False),
  )
  def tail_k(c_hbm, src_hbm, dst_hbm, xt_hbm, tout_hbm,
             sflat, dflat, cflat, xt, acc4):
    c = lax.axis_index("c")
    s = lax.axis_index("s")
    wid = s * NC + c

    def z4(i, carry):
      for l in range(8):
        acc4[i, pl.ds(l * L, L)] = jnp.zeros((L,), F32)
      return carry

    lax.fori_loop(0, TR, z4, 0)
    pltpu.sync_copy(xt_hbm, xt)

    def chunk(j, carry):
      ebase = wid * (nch * ch) + j * ch
      pltpu.sync_copy(src_hbm.at[pl.ds(ebase, ch)], sflat)
      pltpu.sync_copy(dst_hbm.at[pl.ds(ebase, ch)], dflat)
      pltpu.sync_copy(c_hbm.at[pl.ds(ebase, ch)], cflat)
      for g in range(ch // L):
        src16 = sflat[pl.ds(g * L, L)]
        dst16 = dflat[pl.ds(g * L, L)]
        cf16 = cflat[pl.ds(g * L, L)]
        ia = src16 * 4
        ib = dst16 * 4
        row = lax.shift_right_logical(src16, 5)
        col = lax.shift_left(jnp.bitwise_and(src16, 31), 2)
        dx = plsc.load_gather(xt, [ia]) - plsc.load_gather(xt, [ib])
        dy = plsc.load_gather(xt, [ia + 1]) - plsc.load_gather(xt, [ib + 1])
        dz = plsc.load_gather(xt, [ia + 2]) - plsc.load_gather(xt, [ib + 2])
        plsc.addupdate_scatter(acc4, [row, col], dx * cf16)
        plsc.addupdate_scatter(acc4, [row, col + 1], dy * cf16)
        plsc.addupdate_scatter(acc4, [row, col + 2], dz * cf16)
        gid = lax.iota(jnp.int32, L) + (ebase + g * L)
        plsc.addupdate_scatter(acc4, [row, col + 3], jnp.ones((L,), F32),
                               mask=gid < E)
      return carry

    lax.fori_loop(0, nch, chunk, 0)
    pltpu.sync_copy(acc4, tout_hbm.at[pl.ds(wid * TR, TR)])

  return tail_k


def kernel(h, x, src, dst, distances, W_msg1, b_msg1, W_msg2, b_msg2,
           W_h1, b_h1, W_h2, b_h2, W_c1, b_c1, W_c2, b_c2):
  N, D = h.shape
  E = src.shape[0]
  info = plsc.get_sparse_core_info()
  NC, NS = info.num_cores, info.num_subcores
  nworkers = NC * NS
  PER = 128
  K = 2
  CH = PER * K
  epw = EP // nworkers
  nch = epw // CH
  assert EP % nworkers == 0 and epw % CH == 0 and NC == 2

  mesh = plsc.VectorSubcoreMesh(core_axis_name="c", subcore_axis_name="s")

  w1a = W_msg1[:D]
  w1b = W_msg1[D:2 * D]
  wd = W_msg1[2 * D:2 * D + 1]
  b1r = b_msg1.reshape(1, D)
  b2r = b_msg2.reshape(1, D)
  bc1r = b_c1.reshape(1, D)
  wc2r = W_c2.reshape(1, D)
  bc2r = b_c2.reshape(1, 1)
  wha = W_h1[:D]
  whb = W_h1[D:]
  bh1r = b_h1.reshape(1, D)
  bh2r = b_h2.reshape(1, D)
  srcp = jnp.pad(src, (0, EP - E))
  dstp = jnp.pad(dst, (0, EP - E))
  dcol = jnp.pad(distances, (0, EP - E)).reshape(EP, 1)
  xp4 = jnp.pad(x, ((0, 0), (0, 1)))
  xt1d = xp4.reshape(N * 4)
  zerosN = jnp.zeros((N, D), F32)

  NB = 1000
  tables = pl.pallas_call(
      _tables_body,
      grid=(N // NB,),
      in_specs=[
          pl.BlockSpec((NB, D), lambda i: (i, 0)),
          pl.BlockSpec((D, D), lambda i: (0, 0)),
          pl.BlockSpec((D, D), lambda i: (0, 0)),
      ],
      out_specs=[
          pl.BlockSpec((NB, D), lambda i: (i, 0)),
          pl.BlockSpec((NB, D), lambda i: (i, 0)),
      ],
      out_shape=[
          jax.ShapeDtypeStruct((N, D), F32),
          jax.ShapeDtypeStruct((N, D), F32),
      ],
  )
  P, Q = tables(h, w1a, w1b)

  gather_k = _build_gather(D, nch, K, PER, mesh, NC)
  A, B = gather_k(P, Q, srcp, dstp)

  edge = pl.pallas_call(
      functools.partial(_edge_body, E),
      grid=(EP // EB,),
      in_specs=[
          pl.BlockSpec((EB, D), lambda i: (i, 0)),
          pl.BlockSpec((EB, D), lambda i: (i, 0)),
          pl.BlockSpec((EB, 1), lambda i: (i, 0)),
          pl.BlockSpec((1, D), lambda i: (0, 0)),
          pl.BlockSpec((1, D), lambda i: (0, 0)),
          pl.BlockSpec((D, D), lambda i: (0, 0)),
          pl.BlockSpec((1, D), lambda i: (0, 0)),
          pl.BlockSpec((D, D), lambda i: (0, 0)),
          pl.BlockSpec((1, D), lambda i: (0, 0)),
          pl.BlockSpec((1, D), lambda i: (0, 0)),
          pl.BlockSpec((1, 1), lambda i: (0, 0)),
      ],
      out_specs=[
          pl.BlockSpec((EB, D), lambda i: (i, 0)),
          pl.BlockSpec((EB // 128, 128), lambda i: (i, 0)),
      ],
      out_shape=[
          jax.ShapeDtypeStruct((EP, D), F32),
          jax.ShapeDtypeStruct((EP // 128, 128), F32),
      ],
  )
  eo, coef2d = edge(A, B, dcol, wd, b1r, W_msg2, b2r, W_c1, bc1r, wc2r, bc2r)
  coef1d = coef2d.reshape(EP)

  scatter_k = _build_scatter(N, E, D, nch, K, PER, mesh, NC, NS)
  parts = scatter_k(eo, srcp, zerosN)
  tail_k = _build_tail_scatter(N, E, nch, CH, mesh, NC, NS)
  tails = tail_k(coef1d, srcp, dstp, xt1d)
  p0 = parts[:N]
  p1 = parts[N:]
  TR = ((N + 31) // 32 + 7) // 8 * 8
  tailsum = pl.pallas_call(
      _tailsum_body,
      grid=(nworkers,),
      in_specs=[pl.BlockSpec((TR, 128), lambda i: (i, 0))],
      out_specs=pl.BlockSpec((TR, 128), lambda i: (0, 0)),
      out_shape=jax.ShapeDtypeStruct((TR, 128), F32),
  )
  tl = tailsum(tails).reshape(TR * 32, 4)[:N]

  node = pl.pallas_call(
      _node_body,
      grid=(N // NB,),
      in_specs=[
          pl.BlockSpec((NB, D), lambda i: (i, 0)),
          pl.BlockSpec((NB, 4), lambda i: (i, 0)),
          pl.BlockSpec((NB, D), lambda i: (i, 0)),
          pl.BlockSpec((NB, D), lambda i: (i, 0)),
          pl.BlockSpec((NB, 4), lambda i: (i, 0)),
          pl.BlockSpec((D, D), lambda i: (0, 0)),
          pl.BlockSpec((D, D), lambda i: (0, 0)),
          pl.BlockSpec((1, D), lambda i: (0, 0)),
          pl.BlockSpec((D, D), lambda i: (0, 0)),
          pl.BlockSpec((1, D), lambda i: (0, 0)),
      ],
      out_specs=[
          pl.BlockSpec((NB, D), lambda i: (i, 0)),
          pl.BlockSpec((NB, 4), lambda i: (i, 0)),
      ],
      out_shape=[
          jax.ShapeDtypeStruct((N, D), F32),
          jax.ShapeDtypeStruct((N, 4), F32),
      ],
  )
  h_out, xo4 = node(h, xp4, p0, p1, tl, wha, whb, bh1r, W_h2, bh2r)
  return h_out, xo4[:, :3]


# speedup vs baseline: 4.1134x; 1.0735x over previous
import functools

import jax
import jax.numpy as jnp
from jax import lax
from jax.experimental import pallas as pl
from jax.experimental.pallas import tpu as pltpu
from jax.experimental.pallas import tpu_sc as plsc

F32 = jnp.float32
EP = 327680
EB = 4096
L = 16


def _silu(z):
  return z * jax.nn.sigmoid(z)


def _tables_body(h_ref, w1a_ref, w1b_ref, p_ref, q_ref):
  h = h_ref[...]
  p_ref[...] = jnp.dot(h, w1a_ref[...], preferred_element_type=F32)
  q_ref[...] = jnp.dot(h, w1b_ref[...], preferred_element_type=F32)


def _edge_body(E, a_ref, b_ref, d_ref, wd_ref, b1_ref, w2_ref, b2_ref,
               wc1_ref, bc1_ref, wc2_ref, bc2_ref, mo_ref, co_ref):
  i = pl.program_id(0)
  d = d_ref[...]
  m1 = a_ref[...] + b_ref[...] + (d * d) * wd_ref[...] + b1_ref[...]
  mij = _silu(jnp.dot(_silu(m1), w2_ref[...], preferred_element_type=F32)
              + b2_ref[...])
  c1 = _silu(jnp.dot(mij, wc1_ref[...], preferred_element_type=F32)
             + bc1_ref[...])
  coef = jnp.tanh(jnp.sum(c1 * wc2_ref[...], axis=-1, keepdims=True)
                  + bc2_ref[...])
  rid = lax.broadcasted_iota(jnp.int32, (EB, 1), 0) + i * EB
  valid = jnp.where(rid < E, 1.0, 0.0).astype(F32)
  mo_ref[...] = mij * valid
  co_ref[...] = (coef * valid).reshape(EB // 128, 128)


def _tailsum_body(t_ref, o_ref):
  i = pl.program_id(0)

  @pl.when(i == 0)
  def _():
    o_ref[...] = jnp.zeros_like(o_ref)

  o_ref[...] += t_ref[...]


def _node_body(h_ref, xp_ref, p0_ref, p1_ref, tl_ref, wha_ref,
               whb_ref, bh1_ref, wh2_ref, bh2_ref, ho_ref, xo_ref):
  tail = tl_ref[...]
  cnt = tail[:, 3:4]
  cmax = jnp.maximum(cnt, 1.0)
  mi = (p0_ref[...] + p1_ref[...]) / cmax
  h = h_ref[...]
  u = _silu(jnp.dot(h, wha_ref[...], preferred_element_type=F32)
            + jnp.dot(mi, whb_ref[...], preferred_element_type=F32)
            + bh1_ref[...])
  ho_ref[...] = h + jnp.dot(u, wh2_ref[...], preferred_element_type=F32) + bh2_ref[...]
  xo_ref[...] = xp_ref[...] + tail / cmax


def _build_gather(D, nch, ch, mesh, NC):
  NB2 = 2

  @functools.partial(
      pl.kernel,
      out_type=(jax.ShapeDtypeStruct((EP, D), F32),
                jax.ShapeDtypeStruct((EP, D), F32)),
      mesh=mesh,
      scratch_types=[
          [pltpu.VMEM((ch,), jnp.int32) for _ in range(NB2)],
          [pltpu.VMEM((ch,), jnp.int32) for _ in range(NB2)],
          [pltpu.VMEM((ch, D), F32) for _ in range(NB2)],
          [pltpu.VMEM((ch, D), F32) for _ in range(NB2)],
          [pltpu.SemaphoreType.DMA for _ in range(NB2)],
          pltpu.SemaphoreType.DMA,
          [pltpu.SemaphoreType.DMA for _ in range(NB2)],
      ],
      compiler_params=pltpu.CompilerParams(needs_layout_passes=False),
  )
  def gather_k(p_hbm, q_hbm, src_hbm, dst_hbm, oa_hbm, ob_hbm,
               idxs, idxd, bufa, bufb, isems, gsem, wsems):
    wid = lax.axis_index("s") * NC + lax.axis_index("c")
    base0 = wid * nch * ch

    def issue_idx(j, b):
      eb = base0 + j * ch
      pltpu.async_copy(src_hbm.at[pl.ds(eb, ch)], idxs[b], isems[b])
      pltpu.async_copy(dst_hbm.at[pl.ds(eb, ch)], idxd[b], isems[b])

    issue_idx(0, 0)

    def outer(jo, carry):
      for b in range(NB2):
        j = jo * NB2 + b
        nb_ = (b + 1) % NB2

        @pl.when(j + 1 < nch)
        def _():
          issue_idx(j + 1, nb_)

        pltpu.make_async_copy(src_hbm.at[pl.ds(0, ch)], idxs[b],
                              isems[b]).wait()
        pltpu.make_async_copy(dst_hbm.at[pl.ds(0, ch)], idxd[b],
                              isems[b]).wait()

        @pl.when(j >= NB2)
        def _():
          pltpu.make_async_copy(bufa[b], oa_hbm.at[pl.ds(0, ch)],
                                wsems[b]).wait()
          pltpu.make_async_copy(bufb[b], ob_hbm.at[pl.ds(0, ch)],
                                wsems[b]).wait()

        eb = base0 + j * ch
        ga = pltpu.async_copy(p_hbm.at[idxs[b]], bufa[b], gsem)
        gb = pltpu.async_copy(q_hbm.at[idxd[b]], bufb[b], gsem)
        ga.wait()
        gb.wait()
        pltpu.async_copy(bufa[b], oa_hbm.at[pl.ds(eb, ch)], wsems[b])
        pltpu.async_copy(bufb[b], ob_hbm.at[pl.ds(eb, ch)], wsems[b])
      return carry

    lax.fori_loop(0, nch // NB2, outer, 0)
    for b in range(NB2):
      pltpu.make_async_copy(bufa[b], oa_hbm.at[pl.ds(0, ch)], wsems[b]).wait()
      pltpu.make_async_copy(bufb[b], ob_hbm.at[pl.ds(0, ch)], wsems[b]).wait()

  return gather_k


def _build_scatter(N, E, D, nch, ch, mesh, NC, NS):
  NB2 = 2
  ra = (N // NS) // 8 * 8
  rb = N - (NS - 1) * ra

  @functools.partial(
      pl.kernel,
      out_type=jax.ShapeDtypeStruct((NC * N, D), F32),
      mesh=mesh,
      scratch_types=[
          [pltpu.VMEM((ch,), jnp.int32) for _ in range(NB2)],
          [pltpu.VMEM((ch, D), F32) for _ in range(NB2)],
          [pltpu.SemaphoreType.DMA for _ in range(NB2)],
          pltpu.VMEM_SHARED((N, D), F32),
      ],
      compiler_params=pltpu.CompilerParams(needs_layout_passes=False),
  )
  def scatter_k(e_hbm, src_hbm, z_hbm, out_hbm, idxs, bufs, sems, acc_sh):
    c = lax.axis_index("c")
    s = lax.axis_index("s")
    wid = s * NC + c
    base0 = wid * nch * ch

    @pl.when(s < NS - 1)
    def _():
      pltpu.sync_copy(z_hbm.at[pl.ds(s * ra, ra)], acc_sh.at[pl.ds(s * ra, ra)])

    @pl.when(s == NS - 1)
    def _():
      pltpu.sync_copy(z_hbm.at[pl.ds((NS - 1) * ra, rb)],
                      acc_sh.at[pl.ds((NS - 1) * ra, rb)])

    plsc.subcore_barrier()

    def issue_in(j, b):
      eb = base0 + j * ch
      pltpu.async_copy(src_hbm.at[pl.ds(eb, ch)], idxs[b], sems[b])
      pltpu.async_copy(e_hbm.at[pl.ds(eb, ch)], bufs[b], sems[b])

    issue_in(0, 0)

    def outer(jo, carry):
      for b in range(NB2):
        j = jo * NB2 + b
        nb_ = (b + 1) % NB2

        @pl.when(j + 1 < nch)
        def _():
          issue_in(j + 1, nb_)

        pltpu.make_async_copy(src_hbm.at[pl.ds(0, ch)], idxs[b],
                              sems[b]).wait()
        pltpu.make_async_copy(e_hbm.at[pl.ds(0, ch)], bufs[b],
                              sems[b]).wait()
        pltpu.sync_copy(bufs[b], acc_sh.at[idxs[b]], add=True)
      return carry

    lax.fori_loop(0, nch // NB2, outer, 0)
    plsc.subcore_barrier()

    @pl.when(s < NS - 1)
    def _():
      pltpu.sync_copy(acc_sh.at[pl.ds(s * ra, ra)],
                      out_hbm.at[pl.ds(c * N + s * ra, ra)])

    @pl.when(s == NS - 1)
    def _():
      pltpu.sync_copy(acc_sh.at[pl.ds((NS - 1) * ra, rb)],
                      out_hbm.at[pl.ds(c * N + (NS - 1) * ra, rb)])

  return scatter_k


def _build_tail_scatter(N, E, nch, ch, mesh, NC, NS):
  TR = ((N + 31) // 32 + 7) // 8 * 8

  NB2 = 2

  @functools.partial(
      pl.kernel,
      out_type=jax.ShapeDtypeStruct((NC * NS * TR, 128), F32),
      mesh=mesh,
      scratch_types=[
          [pltpu.VMEM((ch,), jnp.int32) for _ in range(NB2)],
          [pltpu.VMEM((ch,), jnp.int32) for _ in range(NB2)],
          [pltpu.VMEM((ch,), F32) for _ in range(NB2)],
          [pltpu.SemaphoreType.DMA for _ in range(NB2)],
          pltpu.VMEM((N * 4,), F32),
          pltpu.VMEM((TR, 128), F32),
      ],
      compiler_params=pltpu.CompilerParams(needs_layout_passes=False),
  )
  def tail_k(c_hbm, src_hbm, dst_hbm, xt_hbm, tout_hbm,
             sflats, dflats, cflats, sems, xt, acc4):
    c = lax.axis_index("c")
    s = lax.axis_index("s")
    wid = s * NC + c
    base0 = wid * nch * ch

    def z4(i, carry):
      for l in range(8):
        acc4[i, pl.ds(l * L, L)] = jnp.zeros((L,), F32)
      return carry

    lax.fori_loop(0, TR, z4, 0)
    pltpu.sync_copy(xt_hbm, xt)

    def issue_in(j, b):
      eb = base0 + j * ch
      pltpu.async_copy(src_hbm.at[pl.ds(eb, ch)], sflats[b], sems[b])
      pltpu.async_copy(dst_hbm.at[pl.ds(eb, ch)], dflats[b], sems[b])
      pltpu.async_copy(c_hbm.at[pl.ds(eb, ch)], cflats[b], sems[b])

    issue_in(0, 0)

    def outer(jo, carry):
      for b in range(NB2):
        j = jo * NB2 + b
        nb_ = (b + 1) % NB2

        @pl.when(j + 1 < nch)
        def _():
          issue_in(j + 1, nb_)

        pltpu.make_async_copy(src_hbm.at[pl.ds(0, ch)], sflats[b],
                              sems[b]).wait()
        pltpu.make_async_copy(dst_hbm.at[pl.ds(0, ch)], dflats[b],
                              sems[b]).wait()
        pltpu.make_async_copy(c_hbm.at[pl.ds(0, ch)], cflats[b],
                              sems[b]).wait()
        body(j, b)
      return carry

    def body(j, b):
      sflat = sflats[b]
      dflat = dflats[b]
      cflat = cflats[b]
      ebase = base0 + j * ch
      for g in range(ch // L):
        src16 = sflat[pl.ds(g * L, L)]
        dst16 = dflat[pl.ds(g * L, L)]
        cf16 = cflat[pl.ds(g * L, L)]
        ia = src16 * 4
        ib = dst16 * 4
        row = lax.shift_right_logical(src16, 5)
        col = lax.shift_left(jnp.bitwise_and(src16, 31), 2)
        dx = plsc.load_gather(xt, [ia]) - plsc.load_gather(xt, [ib])
        dy = plsc.load_gather(xt, [ia + 1]) - plsc.load_gather(xt, [ib + 1])
        dz = plsc.load_gather(xt, [ia + 2]) - plsc.load_gather(xt, [ib + 2])
        plsc.addupdate_scatter(acc4, [row, col], dx * cf16)
        plsc.addupdate_scatter(acc4, [row, col + 1], dy * cf16)
        plsc.addupdate_scatter(acc4, [row, col + 2], dz * cf16)
        gid = lax.iota(jnp.int32, L) + (ebase + g * L)
        plsc.addupdate_scatter(acc4, [row, col + 3], jnp.ones((L,), F32),
                               mask=gid < E)

    lax.fori_loop(0, nch // NB2, outer, 0)
    pltpu.sync_copy(acc4, tout_hbm.at[pl.ds(wid * TR, TR)])

  return tail_k


def kernel(h, x, src, dst, distances, W_msg1, b_msg1, W_msg2, b_msg2,
           W_h1, b_h1, W_h2, b_h2, W_c1, b_c1, W_c2, b_c2):
  N, D = h.shape
  E = src.shape[0]
  info = plsc.get_sparse_core_info()
  NC, NS = info.num_cores, info.num_subcores
  nworkers = NC * NS
  GCH = 128
  TCH = 512
  epw = EP // nworkers
  assert EP % nworkers == 0 and epw % (2 * GCH) == 0 and epw % (2 * TCH) == 0
  assert NC == 2

  mesh = plsc.VectorSubcoreMesh(core_axis_name="c", subcore_axis_name="s")

  w1a = W_msg1[:D]
  w1b = W_msg1[D:2 * D]
  wd = W_msg1[2 * D:2 * D + 1]
  b1r = b_msg1.reshape(1, D)
  b2r = b_msg2.reshape(1, D)
  bc1r = b_c1.reshape(1, D)
  wc2r = W_c2.reshape(1, D)
  bc2r = b_c2.reshape(1, 1)
  wha = W_h1[:D]
  whb = W_h1[D:]
  bh1r = b_h1.reshape(1, D)
  bh2r = b_h2.reshape(1, D)
  srcp = jnp.pad(src, (0, EP - E))
  dstp = jnp.pad(dst, (0, EP - E))
  dcol = jnp.pad(distances, (0, EP - E)).reshape(EP, 1)
  xp4 = jnp.pad(x, ((0, 0), (0, 1)))
  xt1d = xp4.reshape(N * 4)
  zerosN = jnp.zeros((N, D), F32)

  NB = 1000
  tables = pl.pallas_call(
      _tables_body,
      grid=(N // NB,),
      in_specs=[
          pl.BlockSpec((NB, D), lambda i: (i, 0)),
          pl.BlockSpec((D, D), lambda i: (0, 0)),
          pl.BlockSpec((D, D), lambda i: (0, 0)),
      ],
      out_specs=[
          pl.BlockSpec((NB, D), lambda i: (i, 0)),
          pl.BlockSpec((NB, D), lambda i: (i, 0)),
      ],
      out_shape=[
          jax.ShapeDtypeStruct((N, D), F32),
          jax.ShapeDtypeStruct((N, D), F32),
      ],
  )
  P, Q = tables(h, w1a, w1b)

  gather_k = _build_gather(D, epw // GCH, GCH, mesh, NC)
  A, B = gather_k(P, Q, srcp, dstp)

  edge = pl.pallas_call(
      functools.partial(_edge_body, E),
      grid=(EP // EB,),
      in_specs=[
          pl.BlockSpec((EB, D), lambda i: (i, 0)),
          pl.BlockSpec((EB, D), lambda i: (i, 0)),
          pl.BlockSpec((EB, 1), lambda i: (i, 0)),
          pl.BlockSpec((1, D), lambda i: (0, 0)),
          pl.BlockSpec((1, D), lambda i: (0, 0)),
          pl.BlockSpec((D, D), lambda i: (0, 0)),
          pl.BlockSpec((1, D), lambda i: (0, 0)),
          pl.BlockSpec((D, D), lambda i: (0, 0)),
          pl.BlockSpec((1, D), lambda i: (0, 0)),
          pl.BlockSpec((1, D), lambda i: (0, 0)),
          pl.BlockSpec((1, 1), lambda i: (0, 0)),
      ],
      out_specs=[
          pl.BlockSpec((EB, D), lambda i: (i, 0)),
          pl.BlockSpec((EB // 128, 128), lambda i: (i, 0)),
      ],
      out_shape=[
          jax.ShapeDtypeStruct((EP, D), F32),
          jax.ShapeDtypeStruct((EP // 128, 128), F32),
      ],
  )
  eo, coef2d = edge(A, B, dcol, wd, b1r, W_msg2, b2r, W_c1, bc1r, wc2r, bc2r)
  coef1d = coef2d.reshape(EP)

  scatter_k = _build_scatter(N, E, D, epw // GCH, GCH, mesh, NC, NS)
  parts = scatter_k(eo, srcp, zerosN)
  tail_k = _build_tail_scatter(N, E, epw // TCH, TCH, mesh, NC, NS)
  tails = tail_k(coef1d, srcp, dstp, xt1d)
  p0 = parts[:N]
  p1 = parts[N:]
  TR = ((N + 31) // 32 + 7) // 8 * 8
  tailsum = pl.pallas_call(
      _tailsum_body,
      grid=(nworkers,),
      in_specs=[pl.BlockSpec((TR, 128), lambda i: (i, 0))],
      out_specs=pl.BlockSpec((TR, 128), lambda i: (0, 0)),
      out_shape=jax.ShapeDtypeStruct((TR, 128), F32),
  )
  tl = tailsum(tails).reshape(TR * 32, 4)[:N]

  node = pl.pallas_call(
      _node_body,
      grid=(N // NB,),
      in_specs=[
          pl.BlockSpec((NB, D), lambda i: (i, 0)),
          pl.BlockSpec((NB, 4), lambda i: (i, 0)),
          pl.BlockSpec((NB, D), lambda i: (i, 0)),
          pl.BlockSpec((NB, D), lambda i: (i, 0)),
          pl.BlockSpec((NB, 4), lambda i: (i, 0)),
          pl.BlockSpec((D, D), lambda i: (0, 0)),
          pl.BlockSpec((D, D), lambda i: (0, 0)),
          pl.BlockSpec((1, D), lambda i: (0, 0)),
          pl.BlockSpec((D, D), lambda i: (0, 0)),
          pl.BlockSpec((1, D), lambda i: (0, 0)),
      ],
      out_specs=[
          pl.BlockSpec((NB, D), lambda i: (i, 0)),
          pl.BlockSpec((NB, 4), lambda i: (i, 0)),
      ],
      out_shape=[
          jax.ShapeDtypeStruct((N, D), F32),
          jax.ShapeDtypeStruct((N, 4), F32),
      ],
  )
  h_out, xo4 = node(h, xp4, p0, p1, tl, wha, whb, bh1r, W_h2, bh2r)
  return h_out, xo4[:, :3]


# speedup vs baseline: 4.7588x; 1.1569x over previous
import functools

import jax
import jax.numpy as jnp
from jax import lax
from jax.experimental import pallas as pl
from jax.experimental.pallas import tpu as pltpu
from jax.experimental.pallas import tpu_sc as plsc

F32 = jnp.float32
EP = 327680
EB = 4096
L = 16


def _silu(z):
  return z * jax.nn.sigmoid(z)


BF16 = jnp.bfloat16


def _tables_body(h_ref, w1a_ref, w1b_ref, p_ref, q_ref):
  h = h_ref[...]
  p_ref[...] = jnp.dot(h, w1a_ref[...], preferred_element_type=F32)
  q_ref[...] = jnp.dot(h, w1b_ref[...], preferred_element_type=F32)


def _edge_body(E, a_ref, b_ref, d_ref, wd_ref, b1_ref, w2_ref, b2_ref,
               wc1_ref, bc1_ref, wc2_ref, bc2_ref, mo_ref, co_ref):
  i = pl.program_id(0)
  d = d_ref[...]
  m1 = a_ref[...] + b_ref[...] + (d * d) * wd_ref[...] + b1_ref[...]
  mij = _silu(jnp.dot(_silu(m1), w2_ref[...], preferred_element_type=F32)
              + b2_ref[...])
  c1 = _silu(jnp.dot(mij, wc1_ref[...], preferred_element_type=F32)
             + bc1_ref[...])
  coef = jnp.tanh(jnp.sum(c1 * wc2_ref[...], axis=-1, keepdims=True)
                  + bc2_ref[...])
  rid = lax.broadcasted_iota(jnp.int32, (EB, 1), 0) + i * EB
  valid = jnp.where(rid < E, 1.0, 0.0).astype(F32)
  mo_ref[...] = mij * valid
  co_ref[...] = (coef * valid).reshape(EB // 128, 128)


def _tailsum_body(t_ref, o_ref):
  i = pl.program_id(0)

  @pl.when(i == 0)
  def _():
    o_ref[...] = jnp.zeros_like(o_ref)

  o_ref[...] += t_ref[...]


def _node_body(h_ref, xp_ref, p0_ref, p1_ref, tl_ref, wha_ref,
               whb_ref, bh1_ref, wh2_ref, bh2_ref, ho_ref, xo_ref):
  tail = tl_ref[...]
  cnt = tail[:, 3:4]
  cmax = jnp.maximum(cnt, 1.0)
  mi = (p0_ref[...] + p1_ref[...]) / cmax
  h = h_ref[...]
  u = _silu(jnp.dot(h, wha_ref[...], preferred_element_type=F32)
            + jnp.dot(mi, whb_ref[...], preferred_element_type=F32)
            + bh1_ref[...])
  ho_ref[...] = h + jnp.dot(u, wh2_ref[...], preferred_element_type=F32) + bh2_ref[...]
  xo_ref[...] = xp_ref[...] + tail / cmax


def _build_gather(D, nch, ch, mesh, NC):
  NB2 = 2

  NI = 4

  @functools.partial(
      pl.kernel,
      out_type=(jax.ShapeDtypeStruct((EP, D), F32),
                jax.ShapeDtypeStruct((EP, D), F32)),
      mesh=mesh,
      scratch_types=[
          [pltpu.VMEM((ch,), jnp.int32) for _ in range(NI)],
          [pltpu.VMEM((ch,), jnp.int32) for _ in range(NI)],
          [pltpu.VMEM((ch, D), F32) for _ in range(NB2)],
          [pltpu.VMEM((ch, D), F32) for _ in range(NB2)],
          [pltpu.SemaphoreType.DMA for _ in range(NI)],
          [pltpu.SemaphoreType.DMA for _ in range(NB2)],
          [pltpu.SemaphoreType.DMA for _ in range(NB2)],
      ],
      compiler_params=pltpu.CompilerParams(needs_layout_passes=False),
  )
  def gather_k(p_hbm, q_hbm, src_hbm, dst_hbm, oa_hbm, ob_hbm,
               idxs, idxd, bufa, bufb, isems, gsems, wsems):
    wid = lax.axis_index("s") * NC + lax.axis_index("c")
    base0 = wid * nch * ch

    def issue_idx(j, b):
      eb = base0 + j * ch
      pltpu.async_copy(src_hbm.at[pl.ds(eb, ch)], idxs[b], isems[b])
      pltpu.async_copy(dst_hbm.at[pl.ds(eb, ch)], idxd[b], isems[b])

    def drain_gather(bslot):
      pltpu.make_async_copy(p_hbm.at[idxs[0]], bufa[bslot],
                            gsems[bslot]).wait()
      pltpu.make_async_copy(q_hbm.at[idxd[0]], bufb[bslot],
                            gsems[bslot]).wait()

    def issue_write(j, bslot):
      eb = base0 + j * ch
      pltpu.async_copy(bufa[bslot], oa_hbm.at[pl.ds(eb, ch)], wsems[bslot])
      pltpu.async_copy(bufb[bslot], ob_hbm.at[pl.ds(eb, ch)], wsems[bslot])

    def drain_write(bslot):
      pltpu.make_async_copy(bufa[0], oa_hbm.at[pl.ds(0, ch)],
                            wsems[bslot]).wait()
      pltpu.make_async_copy(bufb[0], ob_hbm.at[pl.ds(0, ch)],
                            wsems[bslot]).wait()

    issue_idx(0, 0)

    def outer(jo, carry):
      for b in range(NI):
        j = jo * NI + b
        bs = b % NB2

        @pl.when(j + 1 < nch)
        def _():
          issue_idx(j + 1, (b + 1) % NI)

        pltpu.make_async_copy(src_hbm.at[pl.ds(0, ch)], idxs[b],
                              isems[b]).wait()
        pltpu.make_async_copy(dst_hbm.at[pl.ds(0, ch)], idxd[b],
                              isems[b]).wait()

        @pl.when(j >= 2)
        def _():
          drain_write(bs)

        pltpu.async_copy(p_hbm.at[idxs[b]], bufa[bs], gsems[bs])
        pltpu.async_copy(q_hbm.at[idxd[b]], bufb[bs], gsems[bs])

        @pl.when(j >= 1)
        def _():
          drain_gather((bs + 1) % NB2)
          issue_write(j - 1, (bs + 1) % NB2)
      return carry

    lax.fori_loop(0, nch // NI, outer, 0)
    drain_gather((nch - 1) % NB2)
    issue_write(nch - 1, (nch - 1) % NB2)
    drain_write(0)
    drain_write(1)

  return gather_k


def _build_scatter(N, E, D, nch, ch, mesh, NC, NS):
  NB2 = 2
  ra = (N // NS) // 8 * 8
  rb = N - (NS - 1) * ra

  @functools.partial(
      pl.kernel,
      out_type=jax.ShapeDtypeStruct((NC * N, D), F32),
      mesh=mesh,
      scratch_types=[
          [pltpu.VMEM((ch,), jnp.int32) for _ in range(NB2)],
          [pltpu.VMEM((ch, D), F32) for _ in range(NB2)],
          [pltpu.SemaphoreType.DMA for _ in range(NB2)],
          pltpu.VMEM_SHARED((N, D), F32),
      ],
      compiler_params=pltpu.CompilerParams(needs_layout_passes=False),
  )
  def scatter_k(e_hbm, src_hbm, z_hbm, out_hbm, idxs, bufs, sems, acc_sh):
    c = lax.axis_index("c")
    s = lax.axis_index("s")
    wid = s * NC + c
    base0 = wid * nch * ch

    @pl.when(s < NS - 1)
    def _():
      pltpu.sync_copy(z_hbm.at[pl.ds(s * ra, ra)], acc_sh.at[pl.ds(s * ra, ra)])

    @pl.when(s == NS - 1)
    def _():
      pltpu.sync_copy(z_hbm.at[pl.ds((NS - 1) * ra, rb)],
                      acc_sh.at[pl.ds((NS - 1) * ra, rb)])

    plsc.subcore_barrier()

    def issue_in(j, b):
      eb = base0 + j * ch
      pltpu.async_copy(src_hbm.at[pl.ds(eb, ch)], idxs[b], sems[b])
      pltpu.async_copy(e_hbm.at[pl.ds(eb, ch)], bufs[b], sems[b])

    issue_in(0, 0)

    def outer(jo, carry):
      for b in range(NB2):
        j = jo * NB2 + b
        nb_ = (b + 1) % NB2

        @pl.when(j + 1 < nch)
        def _():
          issue_in(j + 1, nb_)

        pltpu.make_async_copy(src_hbm.at[pl.ds(0, ch)], idxs[b],
                              sems[b]).wait()
        pltpu.make_async_copy(e_hbm.at[pl.ds(0, ch)], bufs[b],
                              sems[b]).wait()
        pltpu.sync_copy(bufs[b], acc_sh.at[idxs[b]], add=True)
      return carry

    lax.fori_loop(0, nch // NB2, outer, 0)
    plsc.subcore_barrier()

    @pl.when(s < NS - 1)
    def _():
      pltpu.sync_copy(acc_sh.at[pl.ds(s * ra, ra)],
                      out_hbm.at[pl.ds(c * N + s * ra, ra)])

    @pl.when(s == NS - 1)
    def _():
      pltpu.sync_copy(acc_sh.at[pl.ds((NS - 1) * ra, rb)],
                      out_hbm.at[pl.ds(c * N + (NS - 1) * ra, rb)])

  return scatter_k


def _build_tail_scatter(N, E, nch, ch, mesh, NC, NS):
  TR = ((N + 31) // 32 + 7) // 8 * 8

  NB2 = 2

  @functools.partial(
      pl.kernel,
      out_type=jax.ShapeDtypeStruct((NC * NS * TR, 128), F32),
      mesh=mesh,
      scratch_types=[
          [pltpu.VMEM((ch,), jnp.int32) for _ in range(NB2)],
          [pltpu.VMEM((ch,), jnp.int32) for _ in range(NB2)],
          [pltpu.VMEM((ch,), F32) for _ in range(NB2)],
          [pltpu.SemaphoreType.DMA for _ in range(NB2)],
          pltpu.VMEM((N * 4,), F32),
          pltpu.VMEM((TR, 128), F32),
      ],
      compiler_params=pltpu.CompilerParams(needs_layout_passes=False),
  )
  def tail_k(c_hbm, src_hbm, dst_hbm, xt_hbm, tout_hbm,
             sflats, dflats, cflats, sems, xt, acc4):
    c = lax.axis_index("c")
    s = lax.axis_index("s")
    wid = s * NC + c
    base0 = wid * nch * ch

    def z4(i, carry):
      for l in range(8):
        acc4[i, pl.ds(l * L, L)] = jnp.zeros((L,), F32)
      return carry

    lax.fori_loop(0, TR, z4, 0)
    pltpu.sync_copy(xt_hbm, xt)

    def issue_in(j, b):
      eb = base0 + j * ch
      pltpu.async_copy(src_hbm.at[pl.ds(eb, ch)], sflats[b], sems[b])
      pltpu.async_copy(dst_hbm.at[pl.ds(eb, ch)], dflats[b], sems[b])
      pltpu.async_copy(c_hbm.at[pl.ds(eb, ch)], cflats[b], sems[b])

    issue_in(0, 0)

    def outer(jo, carry):
      for b in range(NB2):
        j = jo * NB2 + b
        nb_ = (b + 1) % NB2

        @pl.when(j + 1 < nch)
        def _():
          issue_in(j + 1, nb_)

        pltpu.make_async_copy(src_hbm.at[pl.ds(0, ch)], sflats[b],
                              sems[b]).wait()
        pltpu.make_async_copy(dst_hbm.at[pl.ds(0, ch)], dflats[b],
                              sems[b]).wait()
        pltpu.make_async_copy(c_hbm.at[pl.ds(0, ch)], cflats[b],
                              sems[b]).wait()
        body(j, b)
      return carry

    def body(j, b):
      sflat = sflats[b]
      dflat = dflats[b]
      cflat = cflats[b]
      ebase = base0 + j * ch
      for g in range(ch // L):
        src16 = sflat[pl.ds(g * L, L)]
        dst16 = dflat[pl.ds(g * L, L)]
        cf16 = cflat[pl.ds(g * L, L)]
        ia = src16 * 4
        ib = dst16 * 4
        row = lax.shift_right_logical(src16, 5)
        col = lax.shift_left(jnp.bitwise_and(src16, 31), 2)
        dx = plsc.load_gather(xt, [ia]) - plsc.load_gather(xt, [ib])
        dy = plsc.load_gather(xt, [ia + 1]) - plsc.load_gather(xt, [ib + 1])
        dz = plsc.load_gather(xt, [ia + 2]) - plsc.load_gather(xt, [ib + 2])
        plsc.addupdate_scatter(acc4, [row, col], dx * cf16)
        plsc.addupdate_scatter(acc4, [row, col + 1], dy * cf16)
        plsc.addupdate_scatter(acc4, [row, col + 2], dz * cf16)
        gid = lax.iota(jnp.int32, L) + (ebase + g * L)
        plsc.addupdate_scatter(acc4, [row, col + 3], jnp.ones((L,), F32),
                               mask=gid < E)

    lax.fori_loop(0, nch // NB2, outer, 0)
    pltpu.sync_copy(acc4, tout_hbm.at[pl.ds(wid * TR, TR)])

  return tail_k


def kernel(h, x, src, dst, distances, W_msg1, b_msg1, W_msg2, b_msg2,
           W_h1, b_h1, W_h2, b_h2, W_c1, b_c1, W_c2, b_c2):
  N, D = h.shape
  E = src.shape[0]
  info = plsc.get_sparse_core_info()
  NC, NS = info.num_cores, info.num_subcores
  nworkers = NC * NS
  GCH = 128
  TCH = 512
  epw = EP // nworkers
  assert EP % nworkers == 0 and epw % (2 * GCH) == 0 and epw % (2 * TCH) == 0
  assert NC == 2

  mesh = plsc.VectorSubcoreMesh(core_axis_name="c", subcore_axis_name="s")

  w1a = W_msg1[:D]
  w1b = W_msg1[D:2 * D]
  wd = W_msg1[2 * D:2 * D + 1]
  b1r = b_msg1.reshape(1, D)
  b2r = b_msg2.reshape(1, D)
  bc1r = b_c1.reshape(1, D)
  wc2r = W_c2.reshape(1, D)
  bc2r = b_c2.reshape(1, 1)
  wha = W_h1[:D]
  whb = W_h1[D:]
  bh1r = b_h1.reshape(1, D)
  bh2r = b_h2.reshape(1, D)
  srcp = jnp.pad(src, (0, EP - E))
  dstp = jnp.pad(dst, (0, EP - E))
  dcol = jnp.pad(distances, (0, EP - E)).reshape(EP, 1)
  xp4 = jnp.pad(x, ((0, 0), (0, 1)))
  xt1d = xp4.reshape(N * 4)
  zerosN = jnp.zeros((N, D), F32)

  NB = 1000
  tables = pl.pallas_call(
      _tables_body,
      grid=(N // NB,),
      in_specs=[
          pl.BlockSpec((NB, D), lambda i: (i, 0)),
          pl.BlockSpec((D, D), lambda i: (0, 0)),
          pl.BlockSpec((D, D), lambda i: (0, 0)),
      ],
      out_specs=[
          pl.BlockSpec((NB, D), lambda i: (i, 0)),
          pl.BlockSpec((NB, D), lambda i: (i, 0)),
      ],
      out_shape=[
          jax.ShapeDtypeStruct((N, D), F32),
          jax.ShapeDtypeStruct((N, D), F32),
      ],
  )
  P, Q = tables(h, w1a, w1b)

  gather_k = _build_gather(D, epw // GCH, GCH, mesh, NC)
  A, B = gather_k(P, Q, srcp, dstp)

  edge = pl.pallas_call(
      functools.partial(_edge_body, E),
      grid=(EP // EB,),
      in_specs=[
          pl.BlockSpec((EB, D), lambda i: (i, 0)),
          pl.BlockSpec((EB, D), lambda i: (i, 0)),
          pl.BlockSpec((EB, 1), lambda i: (i, 0)),
          pl.BlockSpec((1, D), lambda i: (0, 0)),
          pl.BlockSpec((1, D), lambda i: (0, 0)),
          pl.BlockSpec((D, D), lambda i: (0, 0)),
          pl.BlockSpec((1, D), lambda i: (0, 0)),
          pl.BlockSpec((D, D), lambda i: (0, 0)),
          pl.BlockSpec((1, D), lambda i: (0, 0)),
          pl.BlockSpec((1, D), lambda i: (0, 0)),
          pl.BlockSpec((1, 1), lambda i: (0, 0)),
      ],
      out_specs=[
          pl.BlockSpec((EB, D), lambda i: (i, 0)),
          pl.BlockSpec((EB // 128, 128), lambda i: (i, 0)),
      ],
      out_shape=[
          jax.ShapeDtypeStruct((EP, D), F32),
          jax.ShapeDtypeStruct((EP // 128, 128), F32),
      ],
  )
  eo, coef2d = edge(A, B, dcol, wd, b1r, W_msg2, b2r, W_c1, bc1r, wc2r, bc2r)
  coef1d = coef2d.reshape(EP)

  scatter_k = _build_scatter(N, E, D, epw // GCH, GCH, mesh, NC, NS)
  parts = scatter_k(eo, srcp, zerosN)
  tail_k = _build_tail_scatter(N, E, epw // TCH, TCH, mesh, NC, NS)
  tails = tail_k(coef1d, srcp, dstp, xt1d)
  p0 = parts[:N]
  p1 = parts[N:]
  TR = ((N + 31) // 32 + 7) // 8 * 8
  tailsum = pl.pallas_call(
      _tailsum_body,
      grid=(nworkers,),
      in_specs=[pl.BlockSpec((TR, 128), lambda i: (i, 0))],
      out_specs=pl.BlockSpec((TR, 128), lambda i: (0, 0)),
      out_shape=jax.ShapeDtypeStruct((TR, 128), F32),
  )
  tl = tailsum(tails).reshape(TR * 32, 4)[:N]

  node = pl.pallas_call(
      _node_body,
      grid=(N // NB,),
      in_specs=[
          pl.BlockSpec((NB, D), lambda i: (i, 0)),
          pl.BlockSpec((NB, 4), lambda i: (i, 0)),
          pl.BlockSpec((NB, D), lambda i: (i, 0)),
          pl.BlockSpec((NB, D), lambda i: (i, 0)),
          pl.BlockSpec((NB, 4), lambda i: (i, 0)),
          pl.BlockSpec((D, D), lambda i: (0, 0)),
          pl.BlockSpec((D, D), lambda i: (0, 0)),
          pl.BlockSpec((1, D), lambda i: (0, 0)),
          pl.BlockSpec((D, D), lambda i: (0, 0)),
          pl.BlockSpec((1, D), lambda i: (0, 0)),
      ],
      out_specs=[
          pl.BlockSpec((NB, D), lambda i: (i, 0)),
          pl.BlockSpec((NB, 4), lambda i: (i, 0)),
      ],
      out_shape=[
          jax.ShapeDtypeStruct((N, D), F32),
          jax.ShapeDtypeStruct((N, 4), F32),
      ],
  )
  h_out, xo4 = node(h, xp4, p0, p1, tl, wha, whb, bh1r, W_h2, bh2r)
  return h_out, xo4[:, :3]


# speedup vs baseline: 4.8779x; 1.0250x over previous
import functools

import jax
import jax.numpy as jnp
from jax import lax
from jax.experimental import pallas as pl
from jax.experimental.pallas import tpu as pltpu
from jax.experimental.pallas import tpu_sc as plsc

F32 = jnp.float32
EP = 327680
EB = 4096
L = 16


def _silu(z):
  return z * jax.nn.sigmoid(z)


BF16 = jnp.bfloat16


def _tables_body(h_ref, w1a_ref, w1b_ref, p_ref, q_ref):
  h = h_ref[...]
  p_ref[...] = jnp.dot(h, w1a_ref[...], preferred_element_type=F32)
  q_ref[...] = jnp.dot(h, w1b_ref[...], preferred_element_type=F32)


def _edge_body(E, gofs, a_ref, b_ref, d_ref, wd_ref, b1_ref, w2_ref, b2_ref,
               wc1_ref, bc1_ref, wc2_ref, bc2_ref, mo_ref, co_ref):
  i = pl.program_id(0)
  d = d_ref[...]
  m1 = a_ref[...] + b_ref[...] + (d * d) * wd_ref[...] + b1_ref[...]
  mij = _silu(jnp.dot(_silu(m1), w2_ref[...], preferred_element_type=F32)
              + b2_ref[...])
  c1 = _silu(jnp.dot(mij, wc1_ref[...], preferred_element_type=F32)
             + bc1_ref[...])
  coef = jnp.tanh(jnp.sum(c1 * wc2_ref[...], axis=-1, keepdims=True)
                  + bc2_ref[...])
  rid = lax.broadcasted_iota(jnp.int32, (EB, 1), 0) + i * EB + gofs
  valid = jnp.where(rid < E, 1.0, 0.0).astype(F32)
  mo_ref[...] = mij * valid
  co_ref[...] = (coef * valid).reshape(EB // 128, 128)


def _tailsum_body(t_ref, o_ref):
  i = pl.program_id(0)

  @pl.when(i == 0)
  def _():
    o_ref[...] = jnp.zeros_like(o_ref)

  o_ref[...] += t_ref[...]


def _node_body(h_ref, xp_ref, p0_ref, p1_ref, tl_ref, wha_ref,
               whb_ref, bh1_ref, wh2_ref, bh2_ref, ho_ref, xo_ref):
  tail = tl_ref[...]
  cnt = tail[:, 3:4]
  cmax = jnp.maximum(cnt, 1.0)
  mi = (p0_ref[...] + p1_ref[...]) / cmax
  h = h_ref[...]
  u = _silu(jnp.dot(h, wha_ref[...], preferred_element_type=F32)
            + jnp.dot(mi, whb_ref[...], preferred_element_type=F32)
            + bh1_ref[...])
  ho_ref[...] = h + jnp.dot(u, wh2_ref[...], preferred_element_type=F32) + bh2_ref[...]
  xo_ref[...] = xp_ref[...] + tail / cmax


def _build_gather(D, nch, ch, mesh, NC, EPH, gofs):
  NB2 = 2

  NI = 4

  @functools.partial(
      pl.kernel,
      out_type=(jax.ShapeDtypeStruct((EPH, D), F32),
                jax.ShapeDtypeStruct((EPH, D), F32)),
      mesh=mesh,
      scratch_types=[
          [pltpu.VMEM((ch,), jnp.int32) for _ in range(NI)],
          [pltpu.VMEM((ch,), jnp.int32) for _ in range(NI)],
          [pltpu.VMEM((ch, D), F32) for _ in range(NB2)],
          [pltpu.VMEM((ch, D), F32) for _ in range(NB2)],
          [pltpu.SemaphoreType.DMA for _ in range(NI)],
          [pltpu.SemaphoreType.DMA for _ in range(NB2)],
          [pltpu.SemaphoreType.DMA for _ in range(NB2)],
      ],
      compiler_params=pltpu.CompilerParams(needs_layout_passes=False),
  )
  def gather_k(p_hbm, q_hbm, src_hbm, dst_hbm, oa_hbm, ob_hbm,
               idxs, idxd, bufa, bufb, isems, gsems, wsems):
    wid = lax.axis_index("s") * NC + lax.axis_index("c")
    base0 = wid * nch * ch

    def issue_idx(j, b):
      eb = gofs + base0 + j * ch
      pltpu.async_copy(src_hbm.at[pl.ds(eb, ch)], idxs[b], isems[b])
      pltpu.async_copy(dst_hbm.at[pl.ds(eb, ch)], idxd[b], isems[b])

    def drain_gather(bslot):
      pltpu.make_async_copy(p_hbm.at[idxs[0]], bufa[bslot],
                            gsems[bslot]).wait()
      pltpu.make_async_copy(q_hbm.at[idxd[0]], bufb[bslot],
                            gsems[bslot]).wait()

    def issue_write(j, bslot):
      eb = base0 + j * ch
      pltpu.async_copy(bufa[bslot], oa_hbm.at[pl.ds(eb, ch)], wsems[bslot])
      pltpu.async_copy(bufb[bslot], ob_hbm.at[pl.ds(eb, ch)], wsems[bslot])

    def drain_write(bslot):
      pltpu.make_async_copy(bufa[0], oa_hbm.at[pl.ds(0, ch)],
                            wsems[bslot]).wait()
      pltpu.make_async_copy(bufb[0], ob_hbm.at[pl.ds(0, ch)],
                            wsems[bslot]).wait()

    issue_idx(0, 0)

    def outer(jo, carry):
      for b in range(NI):
        j = jo * NI + b
        bs = b % NB2

        @pl.when(j + 1 < nch)
        def _():
          issue_idx(j + 1, (b + 1) % NI)

        pltpu.make_async_copy(src_hbm.at[pl.ds(0, ch)], idxs[b],
                              isems[b]).wait()
        pltpu.make_async_copy(dst_hbm.at[pl.ds(0, ch)], idxd[b],
                              isems[b]).wait()

        @pl.when(j >= 2)
        def _():
          drain_write(bs)

        pltpu.async_copy(p_hbm.at[idxs[b]], bufa[bs], gsems[bs])
        pltpu.async_copy(q_hbm.at[idxd[b]], bufb[bs], gsems[bs])

        @pl.when(j >= 1)
        def _():
          drain_gather((bs + 1) % NB2)
          issue_write(j - 1, (bs + 1) % NB2)
      return carry

    lax.fori_loop(0, nch // NI, outer, 0)
    drain_gather((nch - 1) % NB2)
    issue_write(nch - 1, (nch - 1) % NB2)
    drain_write(0)
    drain_write(1)

  return gather_k


def _build_scatter(N, E, D, nch, ch, mesh, NC, NS, gofs):
  NB2 = 2
  ra = (N // NS) // 8 * 8
  rb = N - (NS - 1) * ra

  @functools.partial(
      pl.kernel,
      out_type=jax.ShapeDtypeStruct((NC * N, D), F32),
      mesh=mesh,
      scratch_types=[
          [pltpu.VMEM((ch,), jnp.int32) for _ in range(NB2)],
          [pltpu.VMEM((ch, D), F32) for _ in range(NB2)],
          [pltpu.SemaphoreType.DMA for _ in range(NB2)],
          pltpu.VMEM_SHARED((N, D), F32),
      ],
      compiler_params=pltpu.CompilerParams(needs_layout_passes=False),
  )
  def scatter_k(e_hbm, src_hbm, z_hbm, out_hbm, idxs, bufs, sems, acc_sh):
    c = lax.axis_index("c")
    s = lax.axis_index("s")
    wid = s * NC + c
    base0 = wid * nch * ch

    @pl.when(s < NS - 1)
    def _():
      pltpu.sync_copy(z_hbm.at[pl.ds(c * N + s * ra, ra)],
                      acc_sh.at[pl.ds(s * ra, ra)])

    @pl.when(s == NS - 1)
    def _():
      pltpu.sync_copy(z_hbm.at[pl.ds(c * N + (NS - 1) * ra, rb)],
                      acc_sh.at[pl.ds((NS - 1) * ra, rb)])

    plsc.subcore_barrier()

    def issue_in(j, b):
      eb = base0 + j * ch
      pltpu.async_copy(src_hbm.at[pl.ds(gofs + eb, ch)], idxs[b], sems[b])
      pltpu.async_copy(e_hbm.at[pl.ds(eb, ch)], bufs[b], sems[b])

    issue_in(0, 0)

    def outer(jo, carry):
      for b in range(NB2):
        j = jo * NB2 + b
        nb_ = (b + 1) % NB2

        @pl.when(j + 1 < nch)
        def _():
          issue_in(j + 1, nb_)

        pltpu.make_async_copy(src_hbm.at[pl.ds(0, ch)], idxs[b],
                              sems[b]).wait()
        pltpu.make_async_copy(e_hbm.at[pl.ds(0, ch)], bufs[b],
                              sems[b]).wait()
        pltpu.sync_copy(bufs[b], acc_sh.at[idxs[b]], add=True)
      return carry

    lax.fori_loop(0, nch // NB2, outer, 0)
    plsc.subcore_barrier()

    @pl.when(s < NS - 1)
    def _():
      pltpu.sync_copy(acc_sh.at[pl.ds(s * ra, ra)],
                      out_hbm.at[pl.ds(c * N + s * ra, ra)])

    @pl.when(s == NS - 1)
    def _():
      pltpu.sync_copy(acc_sh.at[pl.ds((NS - 1) * ra, rb)],
                      out_hbm.at[pl.ds(c * N + (NS - 1) * ra, rb)])

  return scatter_k


def _build_tail_scatter(N, E, nch, ch, mesh, NC, NS, gofs):
  TR = ((N + 31) // 32 + 7) // 8 * 8

  NB2 = 2

  @functools.partial(
      pl.kernel,
      out_type=jax.ShapeDtypeStruct((NC * NS * TR, 128), F32),
      mesh=mesh,
      scratch_types=[
          [pltpu.VMEM((ch,), jnp.int32) for _ in range(NB2)],
          [pltpu.VMEM((ch,), jnp.int32) for _ in range(NB2)],
          [pltpu.VMEM((ch,), F32) for _ in range(NB2)],
          [pltpu.SemaphoreType.DMA for _ in range(NB2)],
          pltpu.VMEM((N * 4,), F32),
          pltpu.VMEM((TR, 128), F32),
      ],
      compiler_params=pltpu.CompilerParams(needs_layout_passes=False),
  )
  def tail_k(c_hbm, src_hbm, dst_hbm, xt_hbm, tout_hbm,
             sflats, dflats, cflats, sems, xt, acc4):
    c = lax.axis_index("c")
    s = lax.axis_index("s")
    wid = s * NC + c
    base0 = wid * nch * ch

    def z4(i, carry):
      for l in range(8):
        acc4[i, pl.ds(l * L, L)] = jnp.zeros((L,), F32)
      return carry

    lax.fori_loop(0, TR, z4, 0)
    pltpu.sync_copy(xt_hbm, xt)

    def issue_in(j, b):
      eb = base0 + j * ch
      pltpu.async_copy(src_hbm.at[pl.ds(gofs + eb, ch)], sflats[b], sems[b])
      pltpu.async_copy(dst_hbm.at[pl.ds(gofs + eb, ch)], dflats[b], sems[b])
      pltpu.async_copy(c_hbm.at[pl.ds(eb, ch)], cflats[b], sems[b])

    issue_in(0, 0)

    def outer(jo, carry):
      for b in range(NB2):
        j = jo * NB2 + b
        nb_ = (b + 1) % NB2

        @pl.when(j + 1 < nch)
        def _():
          issue_in(j + 1, nb_)

        pltpu.make_async_copy(src_hbm.at[pl.ds(0, ch)], sflats[b],
                              sems[b]).wait()
        pltpu.make_async_copy(dst_hbm.at[pl.ds(0, ch)], dflats[b],
                              sems[b]).wait()
        pltpu.make_async_copy(c_hbm.at[pl.ds(0, ch)], cflats[b],
                              sems[b]).wait()
        body(j, b)
      return carry

    def body(j, b):
      sflat = sflats[b]
      dflat = dflats[b]
      cflat = cflats[b]
      ebase = gofs + base0 + j * ch
      for g in range(ch // L):
        src16 = sflat[pl.ds(g * L, L)]
        dst16 = dflat[pl.ds(g * L, L)]
        cf16 = cflat[pl.ds(g * L, L)]
        ia = src16 * 4
        ib = dst16 * 4
        row = lax.shift_right_logical(src16, 5)
        col = lax.shift_left(jnp.bitwise_and(src16, 31), 2)
        dx = plsc.load_gather(xt, [ia]) - plsc.load_gather(xt, [ib])
        dy = plsc.load_gather(xt, [ia + 1]) - plsc.load_gather(xt, [ib + 1])
        dz = plsc.load_gather(xt, [ia + 2]) - plsc.load_gather(xt, [ib + 2])
        plsc.addupdate_scatter(acc4, [row, col], dx * cf16)
        plsc.addupdate_scatter(acc4, [row, col + 1], dy * cf16)
        plsc.addupdate_scatter(acc4, [row, col + 2], dz * cf16)
        gid = lax.iota(jnp.int32, L) + (ebase + g * L)
        plsc.addupdate_scatter(acc4, [row, col + 3], jnp.ones((L,), F32),
                               mask=gid < E)

    lax.fori_loop(0, nch // NB2, outer, 0)
    pltpu.sync_copy(acc4, tout_hbm.at[pl.ds(wid * TR, TR)])

  return tail_k


def kernel(h, x, src, dst, distances, W_msg1, b_msg1, W_msg2, b_msg2,
           W_h1, b_h1, W_h2, b_h2, W_c1, b_c1, W_c2, b_c2):
  N, D = h.shape
  E = src.shape[0]
  info = plsc.get_sparse_core_info()
  NC, NS = info.num_cores, info.num_subcores
  nworkers = NC * NS
  GCH = 128
  TCH = 512
  epw = EP // nworkers
  assert EP % nworkers == 0 and epw % (2 * GCH) == 0 and epw % (2 * TCH) == 0
  assert NC == 2

  mesh = plsc.VectorSubcoreMesh(core_axis_name="c", subcore_axis_name="s")

  w1a = W_msg1[:D]
  w1b = W_msg1[D:2 * D]
  wd = W_msg1[2 * D:2 * D + 1]
  b1r = b_msg1.reshape(1, D)
  b2r = b_msg2.reshape(1, D)
  bc1r = b_c1.reshape(1, D)
  wc2r = W_c2.reshape(1, D)
  bc2r = b_c2.reshape(1, 1)
  wha = W_h1[:D]
  whb = W_h1[D:]
  bh1r = b_h1.reshape(1, D)
  bh2r = b_h2.reshape(1, D)
  srcp = jnp.pad(src, (0, EP - E))
  dstp = jnp.pad(dst, (0, EP - E))
  dcol = jnp.pad(distances, (0, EP - E)).reshape(EP, 1)
  xp4 = jnp.pad(x, ((0, 0), (0, 1)))
  xt1d = xp4.reshape(N * 4)
  zerosN = jnp.zeros((N, D), F32)

  NB = 1000
  tables = pl.pallas_call(
      _tables_body,
      grid=(N // NB,),
      in_specs=[
          pl.BlockSpec((NB, D), lambda i: (i, 0)),
          pl.BlockSpec((D, D), lambda i: (0, 0)),
          pl.BlockSpec((D, D), lambda i: (0, 0)),
      ],
      out_specs=[
          pl.BlockSpec((NB, D), lambda i: (i, 0)),
          pl.BlockSpec((NB, D), lambda i: (i, 0)),
      ],
      out_shape=[
          jax.ShapeDtypeStruct((N, D), F32),
          jax.ShapeDtypeStruct((N, D), F32),
      ],
  )
  P, Q = tables(h, w1a, w1b)

  EPH = EP // 2
  eph_w = EPH // nworkers

  def edge_call(gofs):
    return pl.pallas_call(
        functools.partial(_edge_body, E, gofs),
        grid=(EPH // EB,),
        in_specs=[
            pl.BlockSpec((EB, D), lambda i: (i, 0)),
            pl.BlockSpec((EB, D), lambda i: (i, 0)),
            pl.BlockSpec((EB, 1), lambda i: (i, 0)),
            pl.BlockSpec((1, D), lambda i: (0, 0)),
            pl.BlockSpec((1, D), lambda i: (0, 0)),
            pl.BlockSpec((D, D), lambda i: (0, 0)),
            pl.BlockSpec((1, D), lambda i: (0, 0)),
            pl.BlockSpec((D, D), lambda i: (0, 0)),
            pl.BlockSpec((1, D), lambda i: (0, 0)),
            pl.BlockSpec((1, D), lambda i: (0, 0)),
            pl.BlockSpec((1, 1), lambda i: (0, 0)),
        ],
        out_specs=[
            pl.BlockSpec((EB, D), lambda i: (i, 0)),
            pl.BlockSpec((EB // 128, 128), lambda i: (i, 0)),
        ],
        out_shape=[
            jax.ShapeDtypeStruct((EPH, D), F32),
            jax.ShapeDtypeStruct((EPH // 128, 128), F32),
        ],
    )

  zeros2N = jnp.zeros((NC * N, D), F32)
  eo = [None, None]
  coef1d = [None, None]
  tails = [None, None]
  AB = [None, None]
  for half in range(2):
    gofs = half * EPH
    gather_k = _build_gather(D, eph_w // GCH, GCH, mesh, NC, EPH, gofs)
    AB[half] = gather_k(P, Q, srcp, dstp)
  for half in range(2):
    gofs = half * EPH
    Ah, Bh = AB[half]
    eoh, c2d = edge_call(gofs)(Ah, Bh, dcol[gofs:gofs + EPH], wd, b1r,
                               W_msg2, b2r, W_c1, bc1r, wc2r, bc2r)
    eo[half] = eoh
    coef1d[half] = c2d.reshape(EPH)

  parts = zeros2N
  for half in range(2):
    gofs = half * EPH
    scatter_k = _build_scatter(N, E, D, eph_w // GCH, GCH, mesh, NC, NS, gofs)
    parts = scatter_k(eo[half], srcp, parts)
    tail_k = _build_tail_scatter(N, E, eph_w // TCH, TCH, mesh, NC, NS, gofs)
    tails[half] = tail_k(coef1d[half], srcp, dstp, xt1d)

  p0 = parts[:N]
  p1 = parts[N:]
  TR = ((N + 31) // 32 + 7) // 8 * 8
  tailsum = pl.pallas_call(
      _tailsum_body,
      grid=(2 * nworkers,),
      in_specs=[pl.BlockSpec((TR, 128), lambda i: (i, 0))],
      out_specs=pl.BlockSpec((TR, 128), lambda i: (0, 0)),
      out_shape=jax.ShapeDtypeStruct((TR, 128), F32),
  )
  tl = tailsum(jnp.concatenate(tails)).reshape(TR * 32, 4)[:N]

  node = pl.pallas_call(
      _node_body,
      grid=(N // NB,),
      in_specs=[
          pl.BlockSpec((NB, D), lambda i: (i, 0)),
          pl.BlockSpec((NB, 4), lambda i: (i, 0)),
          pl.BlockSpec((NB, D), lambda i: (i, 0)),
          pl.BlockSpec((NB, D), lambda i: (i, 0)),
          pl.BlockSpec((NB, 4), lambda i: (i, 0)),
          pl.BlockSpec((D, D), lambda i: (0, 0)),
          pl.BlockSpec((D, D), lambda i: (0, 0)),
          pl.BlockSpec((1, D), lambda i: (0, 0)),
          pl.BlockSpec((D, D), lambda i: (0, 0)),
          pl.BlockSpec((1, D), lambda i: (0, 0)),
      ],
      out_specs=[
          pl.BlockSpec((NB, D), lambda i: (i, 0)),
          pl.BlockSpec((NB, 4), lambda i: (i, 0)),
      ],
      out_shape=[
          jax.ShapeDtypeStruct((N, D), F32),
          jax.ShapeDtypeStruct((N, 4), F32),
      ],
  )
  h_out, xo4 = node(h, xp4, p0, p1, tl, wha, whb, bh1r, W_h2, bh2r)
  return h_out, xo4[:, :3]
